# Initial kernel scaffold; baseline (speedup 1.0000x reference)
import jax
import jax.numpy as jnp
from jax.experimental import pallas as pl


def kernel(logits):
    raise NotImplementedError("write your pallas kernel here")


# speedup vs baseline: 90.9131x; 90.9131x over previous
import jax
import jax.numpy as jnp
from jax.experimental import pallas as pl

_K = 64
_N = 32768
_ROWS_PER_BLOCK = 8
_INT_MIN = -2147483648


def _topk_mask_block(x_ref, o_ref):
    x = x_ref[...]
    i = jax.lax.bitcast_convert_type(x, jnp.int32)
    key = i ^ ((i >> 31) & jnp.int32(0x7FFFFFFF))

    cnt_pos = jnp.sum((key >= 0).astype(jnp.int32), axis=1, keepdims=True)
    t0 = jnp.where(cnt_pos >= _K, jnp.int32(0), jnp.int32(_INT_MIN))

    def body(b, t):
        bit = jnp.int32(1) << (jnp.int32(30) - b)
        cand = t | bit
        cnt = jnp.sum((key >= cand).astype(jnp.int32), axis=1, keepdims=True)
        return jnp.where(cnt >= _K, cand, t)

    thr = jax.lax.fori_loop(0, 31, body, t0)
    o_ref[...] = (key >= thr).astype(jnp.float32)


def kernel(logits):
    x = jnp.squeeze(logits, axis=1)
    b = x.shape[0]
    grid = (b // _ROWS_PER_BLOCK,)
    out = pl.pallas_call(
        _topk_mask_block,
        grid=grid,
        in_specs=[pl.BlockSpec((_ROWS_PER_BLOCK, _N), lambda r: (r, 0))],
        out_specs=pl.BlockSpec((_ROWS_PER_BLOCK, _N), lambda r: (r, 0)),
        out_shape=jax.ShapeDtypeStruct((b, _N), jnp.float32),
    )(x)
    return out
